# baseline (device time: 15255 ns/iter reference)
import jax
import jax.numpy as jnp
from jax import lax
from jax.experimental import pallas as pl
from jax.experimental.pallas import tpu as pltpu

ROWS = 512
CROWS = 64
NC = 5
NF = 3


def kernel(x, pi):
    def body(pi_ref, x_ref, out_ref, fbuf, xs, xr, ys, yr, ls):
        my_x = lax.axis_index("x")
        my_y = lax.axis_index("y")
        my_z = lax.axis_index("z")
        dst_x = pi_ref[my_x]
        other_y = 1 - my_y

        top = ROWS - CROWS
        base = my_y * top
        sign = 1 - 2 * my_y
        nb_base = other_y * top
        nb_sign = 1 - 2 * other_y

        def copy(src_ref, dst_ref, send_sem, recv_sem, device_id):
            return pltpu.make_async_remote_copy(
                src_ref=src_ref, dst_ref=dst_ref,
                send_sem=send_sem, recv_sem=recv_sem,
                device_id=device_id, device_id_type=pl.DeviceIdType.MESH,
            )

        x_dev = (dst_x, my_y, my_z)
        y_dev = (my_x, other_y, my_z)

        barrier_sem = pltpu.get_barrier_semaphore()
        for dev in (x_dev, y_dev):
            pl.semaphore_signal(
                barrier_sem, inc=1,
                device_id=dev, device_id_type=pl.DeviceIdType.MESH,
            )
        pl.semaphore_wait(barrier_sem, 2)

        x_rdmas = []
        for k in range(NF):
            rows = pl.ds(base + sign * (k * CROWS), CROWS)
            r = copy(x_ref.at[0, rows, :], fbuf.at[k],
                     xs.at[k], xr.at[k], x_dev)
            r.start()
            x_rdmas.append(r)
        for k in range(NF, NC):
            rows = pl.ds(base + sign * (k * CROWS), CROWS)
            r = copy(x_ref.at[0, rows, :], out_ref.at[0, rows, :],
                     xs.at[k], xr.at[k], x_dev)
            r.start()
            x_rdmas.append(r)

        y_rdmas, l_dmas = [], []
        for k in range(NF):
            x_rdmas[k].wait_recv()
            rows = pl.ds(base + sign * (k * CROWS), CROWS)
            f = copy(fbuf.at[k], out_ref.at[0, rows, :],
                     ys.at[k], yr.at[k], y_dev)
            f.start()
            y_rdmas.append(f)
            l = pltpu.make_async_copy(
                fbuf.at[k], out_ref.at[0, rows, :], ls.at[k]
            )
            l.start()
            l_dmas.append(l)

        for k in range(NF, NC):
            x_rdmas[k].wait_recv()

        for k in range(NF):
            rows = pl.ds(nb_base + nb_sign * (k * CROWS), CROWS)
            copy(out_ref.at[0, rows, :], out_ref.at[0, rows, :],
                 ys.at[k], yr.at[k], y_dev).wait_recv()

        for k in range(NC):
            x_rdmas[k].wait_send()
        for k in range(NF):
            y_rdmas[k].wait_send()
            l_dmas[k].wait()

    return pl.pallas_call(
        body,
        out_shape=jax.ShapeDtypeStruct(x.shape, x.dtype),
        in_specs=[
            pl.BlockSpec(memory_space=pltpu.SMEM),
            pl.BlockSpec(memory_space=pl.ANY),
        ],
        out_specs=pl.BlockSpec(memory_space=pl.ANY),
        scratch_shapes=[
            pltpu.VMEM((NF, CROWS, 512), jnp.float32),
            pltpu.SemaphoreType.DMA((NC,)),
            pltpu.SemaphoreType.DMA((NC,)),
            pltpu.SemaphoreType.DMA((NF,)),
            pltpu.SemaphoreType.DMA((NF,)),
            pltpu.SemaphoreType.DMA((NF,)),
        ],
        compiler_params=pltpu.CompilerParams(collective_id=0),
    )(pi, x)


# device time: 15045 ns/iter; 1.0140x vs baseline; 1.0140x over previous
import jax
import jax.numpy as jnp
from jax import lax
from jax.experimental import pallas as pl
from jax.experimental.pallas import tpu as pltpu

ROWS = 512
CROWS = 32
NC = 9
NF = 7


def kernel(x, pi):
    def body(pi_ref, x_ref, out_ref, xs, xr, ys, yr):
        my_x = lax.axis_index("x")
        my_y = lax.axis_index("y")
        my_z = lax.axis_index("z")
        dst_x = pi_ref[my_x]
        other_y = 1 - my_y

        top = ROWS - CROWS
        base = my_y * top
        sign = 1 - 2 * my_y
        nb_base = other_y * top
        nb_sign = 1 - 2 * other_y

        def copy(src_ref, dst_ref, send_sem, recv_sem, device_id):
            return pltpu.make_async_remote_copy(
                src_ref=src_ref, dst_ref=dst_ref,
                send_sem=send_sem, recv_sem=recv_sem,
                device_id=device_id, device_id_type=pl.DeviceIdType.MESH,
            )

        x_dev = (dst_x, my_y, my_z)
        y_dev = (my_x, other_y, my_z)

        barrier_sem = pltpu.get_barrier_semaphore()
        for dev in (x_dev, y_dev):
            pl.semaphore_signal(
                barrier_sem, inc=1,
                device_id=dev, device_id_type=pl.DeviceIdType.MESH,
            )
        pl.semaphore_wait(barrier_sem, 2)

        x_rdmas = []
        for k in range(NC):
            rows = pl.ds(base + sign * (k * CROWS), CROWS)
            r = copy(x_ref.at[0, rows, :], out_ref.at[0, rows, :],
                     xs.at[k], xr.at[k], x_dev)
            r.start()
            x_rdmas.append(r)

        y_rdmas = []
        for k in range(NF):
            x_rdmas[k].wait_recv()
            rows = pl.ds(base + sign * (k * CROWS), CROWS)
            f = copy(out_ref.at[0, rows, :], out_ref.at[0, rows, :],
                     ys.at[k], yr.at[k], y_dev)
            f.start()
            y_rdmas.append(f)

        for k in range(NF, NC):
            x_rdmas[k].wait_recv()

        for k in range(NF):
            rows = pl.ds(nb_base + nb_sign * (k * CROWS), CROWS)
            copy(out_ref.at[0, rows, :], out_ref.at[0, rows, :],
                 ys.at[k], yr.at[k], y_dev).wait_recv()

        for k in range(NC):
            x_rdmas[k].wait_send()
        for k in range(NF):
            y_rdmas[k].wait_send()

    return pl.pallas_call(
        body,
        out_shape=jax.ShapeDtypeStruct(x.shape, x.dtype),
        in_specs=[
            pl.BlockSpec(memory_space=pltpu.SMEM),
            pl.BlockSpec(memory_space=pl.ANY),
        ],
        out_specs=pl.BlockSpec(memory_space=pl.ANY),
        scratch_shapes=[
            pltpu.SemaphoreType.DMA((NC,)),
            pltpu.SemaphoreType.DMA((NC,)),
            pltpu.SemaphoreType.DMA((NF,)),
            pltpu.SemaphoreType.DMA((NF,)),
        ],
        compiler_params=pltpu.CompilerParams(collective_id=0),
    )(pi, x)
